# Initial kernel scaffold; baseline (speedup 1.0000x reference)
import jax
import jax.numpy as jnp
from jax import lax
from jax.experimental import pallas as pl
from jax.experimental.pallas import tpu as pltpu
from jax.experimental.pallas import tpu_sc as plsc

VOCAB_ = 1000000
D_ = 64
B_TOTAL_ = 16384 * 50

NC_ = 2
NS_ = 16
NW_ = NC_ * NS_

SUB_ = 128
CHUNK_ = 1024
SUBS_PER_CHUNK_ = CHUNK_ // SUB_
B_PER_W_ = B_TOTAL_ // NW_
CHUNKS_PER_W_ = B_PER_W_ // CHUNK_


def _emb_kernel(table_hbm, idx_hbm, out_hbm, idx_v, rows_v, sem):
    wid = lax.axis_index("s") * NC_ + lax.axis_index("c")
    base = wid * B_PER_W_

    def body(g, _):
        row0 = base + g * CHUNK_
        pltpu.sync_copy(idx_hbm.at[pl.ds(row0 // SUB_, SUBS_PER_CHUNK_)], idx_v)
        copies = []
        for j in range(SUBS_PER_CHUNK_):
            copies.append(
                pltpu.async_copy(
                    table_hbm.at[idx_v.at[j]],
                    rows_v.at[pl.ds(j * SUB_, SUB_)],
                    sem,
                )
            )
        for c in copies:
            c.wait()
        pltpu.sync_copy(rows_v, out_hbm.at[pl.ds(row0, CHUNK_)])
        return ()

    lax.fori_loop(0, CHUNKS_PER_W_, body, (), unroll=False)


@jax.jit
def kernel(token_ids, hidden):
    idx_flat = token_ids.reshape(-1).astype(jnp.int32)
    idx_2d = idx_flat.reshape(B_TOTAL_ // SUB_, SUB_)

    mesh = plsc.VectorSubcoreMesh(core_axis_name="c", subcore_axis_name="s")
    run = pl.kernel(
        _emb_kernel,
        out_type=jax.ShapeDtypeStruct((B_TOTAL_, D_), jnp.float32),
        mesh=mesh,
        scratch_types=[
            pltpu.VMEM((SUBS_PER_CHUNK_, SUB_), jnp.int32),
            pltpu.VMEM((CHUNK_, D_), jnp.float32),
            pltpu.SemaphoreType.DMA,
        ],
    )
    out = run(hidden, idx_2d)
    return out.reshape(token_ids.shape + (D_,))


# speedup vs baseline: 1.8445x; 1.8445x over previous
import jax
import jax.numpy as jnp
from jax import lax
from jax.experimental import pallas as pl
from jax.experimental.pallas import tpu as pltpu
from jax.experimental.pallas import tpu_sc as plsc

VOCAB_ = 1000000
D_ = 64
B_TOTAL_ = 16384 * 50

NC_ = 2
NS_ = 16
NW_ = NC_ * NS_

SUB_ = 128
CHUNK_ = 1024
SUBS_PER_CHUNK_ = CHUNK_ // SUB_
B_PER_W_ = B_TOTAL_ // NW_
CHUNKS_PER_W_ = B_PER_W_ // CHUNK_


def _emb_kernel(table_hbm, idx_hbm, out_hbm, idx_v, rows_v, sem):
    wid = lax.axis_index("s") * NC_ + lax.axis_index("c")
    base = wid * B_PER_W_

    def body(g, _):
        row0 = pl.multiple_of(base + g * CHUNK_, CHUNK_)
        idx_row0 = pl.multiple_of(row0 // SUB_, SUBS_PER_CHUNK_)
        pltpu.sync_copy(idx_hbm.at[pl.ds(idx_row0, SUBS_PER_CHUNK_)], idx_v)
        copies = []
        for j in range(SUBS_PER_CHUNK_):
            copies.append(
                pltpu.async_copy(
                    table_hbm.at[idx_v.at[j]],
                    rows_v.at[pl.ds(j * SUB_, SUB_)],
                    sem,
                )
            )
        for c in copies:
            c.wait()
        pltpu.sync_copy(rows_v, out_hbm.at[pl.ds(row0, CHUNK_)])
        return ()

    lax.fori_loop(0, CHUNKS_PER_W_, body, (), unroll=False)


@jax.jit
def kernel(token_ids, hidden):
    idx_flat = token_ids.reshape(-1).astype(jnp.int32)
    idx_2d = idx_flat.reshape(B_TOTAL_ // SUB_, SUB_)

    mesh = plsc.VectorSubcoreMesh(core_axis_name="c", subcore_axis_name="s")
    run = pl.kernel(
        _emb_kernel,
        out_type=jax.ShapeDtypeStruct((B_TOTAL_, D_), jnp.float32),
        mesh=mesh,
        scratch_types=[
            pltpu.VMEM((SUBS_PER_CHUNK_, SUB_), jnp.int32),
            pltpu.VMEM((CHUNK_, D_), jnp.float32),
            pltpu.SemaphoreType.DMA,
        ],
        compiler_params=pltpu.CompilerParams(use_tc_tiling_on_sc=False),
    )
    out = run(hidden, idx_2d)
    return out.reshape(token_ids.shape + (D_,))


# speedup vs baseline: 1.8728x; 1.0154x over previous
import jax
import jax.numpy as jnp
from jax import lax
from jax.experimental import pallas as pl
from jax.experimental.pallas import tpu as pltpu
from jax.experimental.pallas import tpu_sc as plsc

VOCAB_ = 1000000
D_ = 64
B_TOTAL_ = 16384 * 50

NC_ = 2
NS_ = 16
NW_ = NC_ * NS_

SUB_ = 128
CHUNK_ = 256
SPC_ = CHUNK_ // SUB_
B_PER_W_ = B_TOTAL_ // NW_
N_ = B_PER_W_ // CHUNK_
NBUF_ = 4
P_ = 2
GROUPS_ = N_ // NBUF_


def _emb_kernel(table_hbm, idx_hbm, out_hbm, idx_v, rows_v, *sems):
    gsems = sems[:NBUF_]
    wsems = sems[NBUF_:]
    wid = lax.axis_index("s") * NC_ + lax.axis_index("c")
    cbase = wid * N_
    base = wid * B_PER_W_

    pltpu.sync_copy(idx_hbm.at[pl.ds(cbase, N_)], idx_v)

    def out_slice(t):
        row0 = pl.multiple_of(base + t * CHUNK_, CHUNK_)
        return out_hbm.at[pl.ds(row0, CHUNK_)]

    def fire_gathers(t, b):
        for j in range(SPC_):
            pltpu.async_copy(
                table_hbm.at[idx_v.at[t, j]],
                rows_v.at[b, pl.ds(j * SUB_, SUB_)],
                gsems[b],
            )

    def wait_gathers(t, b):
        pltpu.make_async_copy(out_slice(t), rows_v.at[b], gsems[b]).wait()

    def fire_write(t, b):
        pltpu.async_copy(rows_v.at[b], out_slice(t), wsems[b])

    def wait_write(t, b):
        pltpu.make_async_copy(rows_v.at[b], out_slice(t), wsems[b]).wait()

    def turn(t, b, bf, fire, drain_w):
        if fire:
            if drain_w:
                wait_write(t + P_ - NBUF_, bf)
            fire_gathers(t + P_, bf)
        wait_gathers(t, b)
        fire_write(t, b)

    fire_gathers(0, 0)
    fire_gathers(1, 1)

    turn(0, 0, 2, True, False)
    turn(1, 1, 3, True, False)
    turn(2, 2, 0, True, True)
    turn(3, 3, 1, True, True)

    def body(gg, _):
        t0 = gg * NBUF_
        for b in range(NBUF_):
            turn(t0 + b, b, (b + P_) % NBUF_, True, True)
        return ()

    lax.fori_loop(1, GROUPS_ - 1, body, (), unroll=False)

    tl = N_ - NBUF_
    turn(tl + 0, 0, 2, True, True)
    turn(tl + 1, 1, 3, True, True)
    turn(tl + 2, 2, 0, False, False)
    turn(tl + 3, 3, 1, False, False)

    for b in range(NBUF_):
        wait_write(N_ - NBUF_ + b, b)


@jax.jit
def kernel(token_ids, hidden):
    idx_flat = token_ids.reshape(-1).astype(jnp.int32)
    idx_3d = idx_flat.reshape(B_TOTAL_ // CHUNK_ // 1, SPC_, SUB_).reshape(
        NW_ * N_, SPC_, SUB_
    )

    mesh = plsc.VectorSubcoreMesh(core_axis_name="c", subcore_axis_name="s")
    run = pl.kernel(
        _emb_kernel,
        out_type=jax.ShapeDtypeStruct((B_TOTAL_, D_), jnp.float32),
        mesh=mesh,
        scratch_types=[
            pltpu.VMEM((N_, SPC_, SUB_), jnp.int32),
            pltpu.VMEM((NBUF_, CHUNK_, D_), jnp.float32),
        ]
        + [pltpu.SemaphoreType.DMA] * (2 * NBUF_),
        compiler_params=pltpu.CompilerParams(use_tc_tiling_on_sc=False),
    )
    out = run(hidden, idx_3d)
    return out.reshape(token_ids.shape + (D_,))


# speedup vs baseline: 1.8755x; 1.0014x over previous
import jax
import jax.numpy as jnp
from jax import lax
from jax.experimental import pallas as pl
from jax.experimental.pallas import tpu as pltpu
from jax.experimental.pallas import tpu_sc as plsc

VOCAB_ = 1000000
D_ = 64
B_TOTAL_ = 16384 * 50

NC_ = 2
NS_ = 16
NW_ = NC_ * NS_

SUB_ = 128
CHUNK_ = 256
SPC_ = CHUNK_ // SUB_
B_PER_W_ = B_TOTAL_ // NW_
N_ = B_PER_W_ // CHUNK_
NBUF_ = 4
P_ = 2
GROUPS_ = N_ // NBUF_


def _emb_kernel(table_hbm, idx_hbm, out_hbm, idx_v, rows_v, *sems):
    gsems = sems[:NBUF_]
    wsems = sems[NBUF_:]
    wid = lax.axis_index("s") * NC_ + lax.axis_index("c")
    base = wid * B_PER_W_
    sub_base = wid * (B_PER_W_ // SUB_)

    pltpu.sync_copy(idx_hbm.at[pl.ds(sub_base, N_ * SPC_)], idx_v)

    def out_slice(t):
        row0 = pl.multiple_of(base + t * CHUNK_, CHUNK_)
        return out_hbm.at[pl.ds(row0, CHUNK_)]

    def fire_gathers(t, b):
        for j in range(SPC_):
            pltpu.async_copy(
                table_hbm.at[idx_v.at[t * SPC_ + j]],
                rows_v.at[b, pl.ds(j * SUB_, SUB_)],
                gsems[b],
            )

    def wait_gathers(t, b):
        pltpu.make_async_copy(out_slice(t), rows_v.at[b], gsems[b]).wait()

    def fire_write(t, b):
        pltpu.async_copy(rows_v.at[b], out_slice(t), wsems[b])

    def wait_write(t, b):
        pltpu.make_async_copy(rows_v.at[b], out_slice(t), wsems[b]).wait()

    def turn(t, b, bf, fire, drain_w):
        if fire:
            if drain_w:
                wait_write(t + P_ - NBUF_, bf)
            fire_gathers(t + P_, bf)
        wait_gathers(t, b)
        fire_write(t, b)

    fire_gathers(0, 0)
    fire_gathers(1, 1)

    turn(0, 0, 2, True, False)
    turn(1, 1, 3, True, False)
    turn(2, 2, 0, True, True)
    turn(3, 3, 1, True, True)

    def body(gg, _):
        t0 = gg * NBUF_
        for b in range(NBUF_):
            turn(t0 + b, b, (b + P_) % NBUF_, True, True)
        return ()

    lax.fori_loop(1, GROUPS_ - 1, body, (), unroll=False)

    tl = N_ - NBUF_
    turn(tl + 0, 0, 2, True, True)
    turn(tl + 1, 1, 3, True, True)
    turn(tl + 2, 2, 0, False, False)
    turn(tl + 3, 3, 1, False, False)

    for b in range(NBUF_):
        wait_write(N_ - NBUF_ + b, b)


@jax.jit
def kernel(token_ids, hidden):
    idx_2d = token_ids.reshape(B_TOTAL_ // SUB_, SUB_).astype(jnp.int32)

    mesh = plsc.VectorSubcoreMesh(core_axis_name="c", subcore_axis_name="s")
    run = pl.kernel(
        _emb_kernel,
        out_type=jax.ShapeDtypeStruct((B_TOTAL_, D_), jnp.float32),
        mesh=mesh,
        scratch_types=[
            pltpu.VMEM((N_ * SPC_, SUB_), jnp.int32),
            pltpu.VMEM((NBUF_, CHUNK_, D_), jnp.float32),
        ]
        + [pltpu.SemaphoreType.DMA] * (2 * NBUF_),
        compiler_params=pltpu.CompilerParams(use_tc_tiling_on_sc=False),
    )
    out = run(hidden, idx_2d)
    return out.reshape(token_ids.shape + (D_,))
